# baseline (speedup 1.0000x reference)
import jax
import jax.numpy as jnp
from jax import lax
from jax.experimental import pallas as pl
from jax.experimental.pallas import tpu as pltpu
from jax.experimental.pallas import tpu_sc as plsc

B, C, H, W = 16, 384, 56, 56
HALF = C // 2
NPOS = B * H * W
NW = 32
PR = NPOS // NW
CR = 16
NCHUNK = PR // CR
NSLOT = 7
NITER = NCHUNK // NSLOT
NVEC = C // 16


def _body(a_hbm, b_hbm, out_hbm, abuf, bbuf, *sems):
    asems = sems[0:NSLOT]
    bsems = sems[NSLOT:2 * NSLOT]
    wsems = sems[2 * NSLOT:3 * NSLOT]
    cid = lax.axis_index("c")
    sid = lax.axis_index("s")
    wid = cid * 16 + sid
    row0 = wid * PR

    lane2 = 2 * lax.iota(jnp.int32, 16)
    cols_a = [32 * j + lane2 for j in range(NVEC // 2)]
    cols_b = [32 * j + lane2 + 1 for j in range(NVEC // 2)]

    def in_copies(k, s):
        base = row0 + k * CR
        return (
            pltpu.make_async_copy(a_hbm.at[pl.ds(base, CR)], abuf.at[s], asems[s]),
            pltpu.make_async_copy(b_hbm.at[pl.ds(base, CR)], bbuf.at[s], bsems[s]),
        )

    def out_copy(k, s):
        base = row0 + k * CR
        return pltpu.make_async_copy(
            abuf.at[s], out_hbm.at[pl.ds(base, CR)], wsems[s])

    def compute(s):
        av = abuf.at[s]
        bv = bbuf.at[s]

        def row_body(r, carry):
            for u in range(2):
                rr = 2 * r + u
                rsplat = jnp.full((16,), rr, jnp.int32)
                for j in range(NVEC // 2):
                    av[rr, pl.ds(16 * j, 16)] = plsc.load_gather(
                        av, [rsplat, cols_a[j]])
                for j in range(NVEC // 2):
                    av[rr, pl.ds(HALF + 16 * j, 16)] = plsc.load_gather(
                        bv, [rsplat, cols_b[j]])
            return carry

        lax.fori_loop(0, CR // 2, row_body, 0)

    for j in range(NSLOT - 1):
        for cp in in_copies(j, j):
            cp.start()

    def ring_body(i, carry):
        k0 = i * NSLOT
        for s in range(NSLOT):
            k = k0 + s
            nxt = k + NSLOT - 1
            @pl.when(nxt < NCHUNK)
            def _():
                @pl.when(nxt >= NSLOT)
                def _():
                    out_copy(nxt - NSLOT, (s + NSLOT - 1) % NSLOT).wait()
                for cp in in_copies(nxt, (s + NSLOT - 1) % NSLOT):
                    cp.start()

            for cp in in_copies(k, s):
                cp.wait()

            compute(s)
            out_copy(k, s).start()
        return carry

    lax.fori_loop(0, NITER, ring_body, 0)
    for s in range(NSLOT):
        k = NCHUNK - NSLOT + s
        out_copy(k, k % NSLOT).wait()


@jax.jit
def _frozen_adder(a2, b2):
    mesh = plsc.VectorSubcoreMesh(core_axis_name="c", subcore_axis_name="s")
    return pl.kernel(
        _body,
        out_type=jax.ShapeDtypeStruct((NPOS, C), jnp.float32),
        mesh=mesh,
        scratch_types=[
            pltpu.VMEM((NSLOT, CR, C), jnp.float32),
            pltpu.VMEM((NSLOT, CR, C), jnp.float32),
        ] + [pltpu.SemaphoreType.DMA] * (3 * NSLOT),
        compiler_params=pltpu.CompilerParams(
            use_tc_tiling_on_sc=True, needs_layout_passes=False),
    )(a2, b2)


def kernel(input_a, input_b):
    a2 = input_a.transpose(0, 2, 3, 1).reshape(NPOS, C)
    b2 = input_b.transpose(0, 2, 3, 1).reshape(NPOS, C)
    out = _frozen_adder(a2, b2)
    return out.reshape(B, H, W, C).transpose(0, 3, 1, 2)


# speedup vs baseline: 1.1800x; 1.1800x over previous
import jax
import jax.numpy as jnp
from jax import lax
from jax.experimental import pallas as pl
from jax.experimental.pallas import tpu as pltpu
from jax.experimental.pallas import tpu_sc as plsc

B, C, H, W = 16, 384, 56, 56
HALF = C // 2
NPOS = B * H * W
NW = 32
PR = NPOS // NW
CR = 32
NCHUNK = PR // CR
NSLOT = 2
NITER = NCHUNK // NSLOT
NVEC = C // 16


def _body(a_hbm, b_hbm, out_hbm, abuf, bbuf, obuf, *sems):
    asems = sems[0:NSLOT]
    bsems = sems[NSLOT:2 * NSLOT]
    wsems = sems[2 * NSLOT:3 * NSLOT]
    cid = lax.axis_index("c")
    sid = lax.axis_index("s")
    wid = cid * 16 + sid
    row0 = wid * PR

    lane2 = 2 * lax.iota(jnp.int32, 16)
    cols_a = [32 * j + lane2 for j in range(NVEC // 2)]
    cols_b = [32 * j + lane2 + 1 for j in range(NVEC // 2)]

    def in_copies(k, s):
        base = row0 + k * CR
        return (
            pltpu.make_async_copy(a_hbm.at[pl.ds(base, CR)], abuf.at[s], asems[s]),
            pltpu.make_async_copy(b_hbm.at[pl.ds(base, CR)], bbuf.at[s], bsems[s]),
        )

    def out_copy(k, s):
        base = row0 + k * CR
        return pltpu.make_async_copy(
            obuf.at[s], out_hbm.at[pl.ds(base, CR)], wsems[s])

    def compute(s):
        av = abuf.at[s]
        bv = bbuf.at[s]
        ov = obuf.at[s]

        def row_body(r, carry):
            for u in range(2):
                rr = 2 * r + u
                rsplat = jnp.full((16,), rr, jnp.int32)
                for j in range(NVEC // 2):
                    ov[rr, pl.ds(16 * j, 16)] = plsc.load_gather(
                        av, [rsplat, cols_a[j]])
                for j in range(NVEC // 2):
                    ov[rr, pl.ds(HALF + 16 * j, 16)] = plsc.load_gather(
                        bv, [rsplat, cols_b[j]])
            return carry

        lax.fori_loop(0, CR // 2, row_body, 0)

    for cp in in_copies(0, 0):
        cp.start()

    def ring_body(i, carry):
        k0 = i * NSLOT
        for s in range(NSLOT):
            k = k0 + s
            for cp in in_copies(k, s):
                cp.wait()

            nxt = k + NSLOT - 1
            @pl.when(nxt < NCHUNK)
            def _():
                for cp in in_copies(nxt, (s + NSLOT - 1) % NSLOT):
                    cp.start()

            @pl.when(i > 0)
            def _():
                out_copy(k - NSLOT, s).wait()

            compute(s)
            out_copy(k, s).start()
        return carry

    lax.fori_loop(0, NITER, ring_body, 0)
    for cp in in_copies(NCHUNK - 1, 0):
        cp.wait()
    out_copy(NCHUNK - 3, 0).wait()
    compute(0)
    out_copy(NCHUNK - 1, 0).start()
    out_copy(NCHUNK - 2, 1).wait()
    out_copy(NCHUNK - 1, 0).wait()


@jax.jit
def _frozen_adder(a2, b2):
    mesh = plsc.VectorSubcoreMesh(core_axis_name="c", subcore_axis_name="s")
    return pl.kernel(
        _body,
        out_type=jax.ShapeDtypeStruct((NPOS, C), jnp.float32),
        mesh=mesh,
        scratch_types=[
            pltpu.VMEM((NSLOT, CR, C), jnp.float32),
            pltpu.VMEM((NSLOT, CR, C), jnp.float32),
            pltpu.VMEM((NSLOT, CR, C), jnp.float32),
        ] + [pltpu.SemaphoreType.DMA] * (3 * NSLOT),
        compiler_params=pltpu.CompilerParams(
            use_tc_tiling_on_sc=True, needs_layout_passes=False),
    )(a2, b2)


def kernel(input_a, input_b):
    a2 = input_a.transpose(0, 2, 3, 1).reshape(NPOS, C)
    b2 = input_b.transpose(0, 2, 3, 1).reshape(NPOS, C)
    out = _frozen_adder(a2, b2)
    return out.reshape(B, H, W, C).transpose(0, 3, 1, 2)
